# initial kernel scaffold (unmeasured)
import jax
import jax.numpy as jnp
from jax import lax
from jax.experimental import pallas as pl
from jax.experimental.pallas import tpu as pltpu

N_DEV = 4
B_LOC = 2
SQ = 512
SKV = 512
HG = 8
DH = 64
DM = 768
DG = HG * DH

NEG = -1e9


def kernel(x, Wq, K_ext, V_ext, Wo):
    def body(x_ref, wq_ref, k_hbm, v_hbm, wo_ref, out_ref,
             wq_s, wo_s, kg_ref, vg_ref, ctx_ref,
             wq_send, wq_recv, wo_send, wo_recv, kv_sem):
        my_i = lax.axis_index("i")
        left = (my_i + N_DEV - 1) % N_DEV
        right = (my_i + 1) % N_DEV

        barrier = pltpu.get_barrier_semaphore()
        for nbr in (left, right):
            pl.semaphore_signal(barrier, inc=1, device_id=(nbr,),
                                device_id_type=pl.DeviceIdType.MESH)
        pl.semaphore_wait(barrier, 2)

        wq_s[0] = wq_ref[...].astype(jnp.bfloat16)
        wo_s[0] = wo_ref[...].astype(jnp.bfloat16)

        for h in range(1, N_DEV):
            cp_q = pltpu.make_async_remote_copy(
                src_ref=wq_s.at[h - 1], dst_ref=wq_s.at[h],
                send_sem=wq_send.at[h - 1], recv_sem=wq_recv.at[h - 1],
                device_id=(right,), device_id_type=pl.DeviceIdType.MESH)
            cp_o = pltpu.make_async_remote_copy(
                src_ref=wo_s.at[h - 1], dst_ref=wo_s.at[h],
                send_sem=wo_send.at[h - 1], recv_sem=wo_recv.at[h - 1],
                device_id=(right,), device_id_type=pl.DeviceIdType.MESH)
            cp_q.start()
            cp_o.start()
            cp_q.wait()
            cp_o.wait()

        qi = lax.broadcasted_iota(jnp.int32, (SQ, SKV), 0)
        ki = lax.broadcasted_iota(jnp.int32, (SQ, SKV), 1)
        mask = (jnp.abs(qi - ki) <= 128) | (ki < 32) | (qi < 32)

        for s in range(N_DEV):
            g = (my_i + N_DEV - s) % N_DEV
            for b in range(B_LOC):
                bg = my_i * B_LOC + b
                copies = []
                for hh in range(HG):
                    h_idx = g * HG + hh
                    ck = pltpu.make_async_copy(
                        k_hbm.at[bg, :, h_idx, :], kg_ref.at[hh],
                        kv_sem.at[0])
                    cv = pltpu.make_async_copy(
                        v_hbm.at[bg, :, h_idx, :], vg_ref.at[hh],
                        kv_sem.at[1])
                    ck.start()
                    cv.start()
                    copies.append((ck, cv))

                xb = x_ref[b].astype(jnp.bfloat16)
                q = lax.dot_general(
                    xb, wq_s[s], (((1,), (0,)), ((), ())),
                    preferred_element_type=jnp.float32)
                q = (q * 0.125).astype(jnp.bfloat16)

                for ck, cv in copies:
                    ck.wait()
                    cv.wait()

                for hh in range(HG):
                    qh = q[:, hh * DH:(hh + 1) * DH]
                    kh = kg_ref[hh].astype(jnp.bfloat16)
                    scores = lax.dot_general(
                        qh, kh, (((1,), (1,)), ((), ())),
                        preferred_element_type=jnp.float32)
                    scores = jnp.where(mask, scores, NEG)
                    m = jnp.max(scores, axis=1, keepdims=True)
                    w = jnp.exp(scores - m)
                    w = w / jnp.sum(w, axis=1, keepdims=True)
                    vh = vg_ref[hh].astype(jnp.bfloat16)
                    ctxh = lax.dot_general(
                        w.astype(jnp.bfloat16), vh,
                        (((1,), (0,)), ((), ())),
                        preferred_element_type=jnp.float32)
                    ctx_ref[:, hh * DH:(hh + 1) * DH] = ctxh.astype(
                        jnp.bfloat16)

                contrib = lax.dot_general(
                    ctx_ref[...], wo_s[s], (((1,), (0,)), ((), ())),
                    preferred_element_type=jnp.float32)
                if s == 0:
                    out_ref[b] = contrib
                else:
                    out_ref[b] = out_ref[b] + contrib

    return pl.pallas_call(
        body,
        out_shape=jax.ShapeDtypeStruct((B_LOC, SQ, DM), jnp.float32),
        in_specs=[
            pl.BlockSpec(memory_space=pltpu.VMEM),
            pl.BlockSpec(memory_space=pltpu.VMEM),
            pl.BlockSpec(memory_space=pltpu.ANY),
            pl.BlockSpec(memory_space=pltpu.ANY),
            pl.BlockSpec(memory_space=pltpu.VMEM),
        ],
        out_specs=pl.BlockSpec(memory_space=pltpu.VMEM),
        scratch_shapes=[
            pltpu.VMEM((N_DEV, DM, DG), jnp.bfloat16),
            pltpu.VMEM((N_DEV, DG, DM), jnp.bfloat16),
            pltpu.VMEM((HG, SKV, DH), jnp.float32),
            pltpu.VMEM((HG, SKV, DH), jnp.float32),
            pltpu.VMEM((SQ, DG), jnp.bfloat16),
            pltpu.SemaphoreType.DMA((N_DEV - 1,)),
            pltpu.SemaphoreType.DMA((N_DEV - 1,)),
            pltpu.SemaphoreType.DMA((N_DEV - 1,)),
            pltpu.SemaphoreType.DMA((N_DEV - 1,)),
            pltpu.SemaphoreType.DMA((2,)),
        ],
        compiler_params=pltpu.CompilerParams(collective_id=0),
    )(x, Wq, K_ext, V_ext, Wo)


# baseline (device time: 244720 ns/iter reference)
import jax
import jax.numpy as jnp
from jax import lax
from jax.experimental import pallas as pl
from jax.experimental.pallas import tpu as pltpu

N_DEV = 4
B_LOC = 2
SQ = 512
SKV = 512
HG = 8
DH = 64
DM = 768
DG = HG * DH

NEG = -1e9


def kernel(x, Wq, K_ext, V_ext, Wo):
    def body(x_ref, wq_ref, k_hbm, v_hbm, wo_ref, out_ref,
             wq_s, wo_s, kg_ref, vg_ref, ctx_ref,
             wq_send, wq_recv, wo_send, wo_recv, kv_sem):
        my_i = lax.axis_index("i")
        left = (my_i + N_DEV - 1) % N_DEV
        right = (my_i + 1) % N_DEV

        barrier = pltpu.get_barrier_semaphore()
        for nbr in (left, right):
            pl.semaphore_signal(barrier, inc=1, device_id=(nbr,),
                                device_id_type=pl.DeviceIdType.MESH)
        pl.semaphore_wait(barrier, 2)

        wq_s[0] = wq_ref[...].astype(jnp.bfloat16)
        wo_s[0] = wo_ref[...].astype(jnp.bfloat16)

        for h in range(1, N_DEV):
            cp_q = pltpu.make_async_remote_copy(
                src_ref=wq_s.at[h - 1], dst_ref=wq_s.at[h],
                send_sem=wq_send.at[h - 1], recv_sem=wq_recv.at[h - 1],
                device_id=(right,), device_id_type=pl.DeviceIdType.MESH)
            cp_o = pltpu.make_async_remote_copy(
                src_ref=wo_s.at[h - 1], dst_ref=wo_s.at[h],
                send_sem=wo_send.at[h - 1], recv_sem=wo_recv.at[h - 1],
                device_id=(right,), device_id_type=pl.DeviceIdType.MESH)
            cp_q.start()
            cp_o.start()
            cp_q.wait()
            cp_o.wait()

        qi = lax.broadcasted_iota(jnp.int32, (SQ, SKV), 0)
        ki = lax.broadcasted_iota(jnp.int32, (SQ, SKV), 1)
        mask = (jnp.abs(qi - ki) <= 128) | (ki < 32) | (qi < 32)

        for s in range(N_DEV):
            g = (my_i + N_DEV - s) % N_DEV
            for b in range(B_LOC):
                bg = my_i * B_LOC + b
                copies = []
                for hh in range(HG):
                    h_idx = g * HG + hh
                    ck = pltpu.make_async_copy(
                        k_hbm.at[bg, :, h_idx, :], kg_ref.at[hh],
                        kv_sem.at[0])
                    cv = pltpu.make_async_copy(
                        v_hbm.at[bg, :, h_idx, :], vg_ref.at[hh],
                        kv_sem.at[1])
                    ck.start()
                    cv.start()
                    copies.append((ck, cv))

                xb = x_ref[b].astype(jnp.bfloat16)
                q = lax.dot_general(
                    xb, wq_s[s], (((1,), (0,)), ((), ())),
                    preferred_element_type=jnp.float32)
                q = (q * 0.125).astype(jnp.bfloat16)

                for ck, cv in copies:
                    ck.wait()
                    cv.wait()

                for hh in range(HG):
                    qh = q[:, hh * DH:(hh + 1) * DH]
                    kh = kg_ref[hh].astype(jnp.bfloat16)
                    scores = lax.dot_general(
                        qh, kh, (((1,), (1,)), ((), ())),
                        preferred_element_type=jnp.float32)
                    scores = jnp.where(mask, scores, NEG)
                    m = jnp.max(scores, axis=1, keepdims=True)
                    w = jnp.exp(scores - m)
                    w = w / jnp.sum(w, axis=1, keepdims=True)
                    vh = vg_ref[hh].astype(jnp.bfloat16)
                    ctxh = lax.dot_general(
                        w.astype(jnp.bfloat16), vh,
                        (((1,), (0,)), ((), ())),
                        preferred_element_type=jnp.float32)
                    ctx_ref[:, hh * DH:(hh + 1) * DH] = ctxh.astype(
                        jnp.bfloat16)

                contrib = lax.dot_general(
                    ctx_ref[...], wo_s[s], (((1,), (0,)), ((), ())),
                    preferred_element_type=jnp.float32)
                if s == 0:
                    out_ref[b] = contrib
                else:
                    out_ref[b] = out_ref[b] + contrib

    return pl.pallas_call(
        body,
        out_shape=jax.ShapeDtypeStruct((B_LOC, SQ, DM), jnp.float32),
        in_specs=[
            pl.BlockSpec(memory_space=pltpu.VMEM),
            pl.BlockSpec(memory_space=pltpu.VMEM),
            pl.BlockSpec(memory_space=pltpu.MemorySpace.HBM),
            pl.BlockSpec(memory_space=pltpu.MemorySpace.HBM),
            pl.BlockSpec(memory_space=pltpu.VMEM),
        ],
        out_specs=pl.BlockSpec(memory_space=pltpu.VMEM),
        scratch_shapes=[
            pltpu.VMEM((N_DEV, DM, DG), jnp.bfloat16),
            pltpu.VMEM((N_DEV, DG, DM), jnp.bfloat16),
            pltpu.VMEM((HG, SKV, DH), jnp.float32),
            pltpu.VMEM((HG, SKV, DH), jnp.float32),
            pltpu.VMEM((SQ, DG), jnp.bfloat16),
            pltpu.SemaphoreType.DMA((N_DEV - 1,)),
            pltpu.SemaphoreType.DMA((N_DEV - 1,)),
            pltpu.SemaphoreType.DMA((N_DEV - 1,)),
            pltpu.SemaphoreType.DMA((N_DEV - 1,)),
            pltpu.SemaphoreType.DMA((2,)),
        ],
        compiler_params=pltpu.CompilerParams(collective_id=0),
    )(x, Wq, K_ext, V_ext, Wo)


# device time: 197962 ns/iter; 1.2362x vs baseline; 1.2362x over previous
import jax
import jax.numpy as jnp
from jax import lax
from jax.experimental import pallas as pl
from jax.experimental.pallas import tpu as pltpu

N_DEV = 4
B_LOC = 2
SQ = 512
SKV = 512
HG = 8
DH = 64
DM = 768
DG = HG * DH
N_STEP = N_DEV * B_LOC

NEG = -1e9


def kernel(x, Wq, K_ext, V_ext, Wo):
    x = x.astype(jnp.bfloat16)
    Wq = Wq.astype(jnp.bfloat16)
    Wo = Wo.astype(jnp.bfloat16)

    def body(x_ref, wq_ref, k_hbm, v_hbm, wo_ref, out_ref,
             wq_s, wo_s, kg_ref, vg_ref,
             wq_send, wq_recv, wo_send, wo_recv, ksem, vsem):
        my_i = lax.axis_index("i")
        left = (my_i + N_DEV - 1) % N_DEV
        right = (my_i + 1) % N_DEV

        barrier = pltpu.get_barrier_semaphore()
        for nbr in (left, right):
            pl.semaphore_signal(barrier, inc=1, device_id=(nbr,),
                                device_id_type=pl.DeviceIdType.MESH)
        pl.semaphore_wait(barrier, 2)

        def start_kv(step):
            s, b = divmod(step, B_LOC)
            g = (my_i + N_DEV - s) % N_DEV
            bg = my_i * B_LOC + b
            p = step % 2
            cps = []
            for hh in range(HG):
                h_idx = g * HG + hh
                ck = pltpu.make_async_copy(
                    k_hbm.at[bg, :, h_idx, :], kg_ref.at[p, hh], ksem.at[p])
                cv = pltpu.make_async_copy(
                    v_hbm.at[bg, :, h_idx, :], vg_ref.at[p, hh], vsem.at[p])
                ck.start()
                cv.start()
                cps.append((ck, cv))
            return cps

        pending = {0: start_kv(0)}

        wq_s[0] = wq_ref[...].T.reshape(HG, DH, DM)
        wo_s[0] = wo_ref[...].reshape(HG, DH, DM)

        def make_hop(h):
            cp_q = pltpu.make_async_remote_copy(
                src_ref=wq_s.at[h - 1], dst_ref=wq_s.at[h],
                send_sem=wq_send.at[h - 1], recv_sem=wq_recv.at[h - 1],
                device_id=(right,), device_id_type=pl.DeviceIdType.MESH)
            cp_o = pltpu.make_async_remote_copy(
                src_ref=wo_s.at[h - 1], dst_ref=wo_s.at[h],
                send_sem=wo_send.at[h - 1], recv_sem=wo_recv.at[h - 1],
                device_id=(right,), device_id_type=pl.DeviceIdType.MESH)
            cp_q.start()
            cp_o.start()
            return (cp_q, cp_o)

        hops = {1: make_hop(1)}

        qi = lax.broadcasted_iota(jnp.int32, (SQ, SKV), 0)
        ki = lax.broadcasted_iota(jnp.int32, (SQ, SKV), 1)
        mask = (jnp.abs(qi - ki) <= 128) | (ki < 32) | (qi < 32)
        maskf = jnp.where(mask, 0.0, NEG).astype(jnp.float32)

        for step in range(N_STEP):
            s, b = divmod(step, B_LOC)
            p = step % 2
            if b == 0 and s >= 1:
                cp_q, cp_o = hops[s]
                cp_q.wait()
                cp_o.wait()
                if s + 1 < N_DEV:
                    hops[s + 1] = make_hop(s + 1)
            for ck, cv in pending.pop(step):
                ck.wait()
                cv.wait()
            if step + 1 < N_STEP:
                pending[step + 1] = start_kv(step + 1)

            xb = x_ref[b]
            step_acc = None
            for hh in range(HG):
                qh = lax.dot_general(
                    xb, wq_s[s, hh], (((1,), (1,)), ((), ())),
                    preferred_element_type=jnp.float32)
                qh = (qh * 0.125).astype(jnp.bfloat16)
                kh = kg_ref[p, hh].astype(jnp.bfloat16)
                scores = lax.dot_general(
                    qh, kh, (((1,), (1,)), ((), ())),
                    preferred_element_type=jnp.float32)
                w = jnp.exp(scores + maskf)
                wsum = jnp.sum(w, axis=1, keepdims=True)
                vh = vg_ref[p, hh].astype(jnp.bfloat16)
                ctx = lax.dot_general(
                    w.astype(jnp.bfloat16), vh,
                    (((1,), (0,)), ((), ())),
                    preferred_element_type=jnp.float32)
                ctx = (ctx / wsum).astype(jnp.bfloat16)
                part = lax.dot_general(
                    ctx, wo_s[s, hh], (((1,), (0,)), ((), ())),
                    preferred_element_type=jnp.float32)
                step_acc = part if step_acc is None else step_acc + part
            if s == 0:
                out_ref[b] = step_acc
            else:
                out_ref[b] = out_ref[b] + step_acc

    return pl.pallas_call(
        body,
        out_shape=jax.ShapeDtypeStruct((B_LOC, SQ, DM), jnp.float32),
        in_specs=[
            pl.BlockSpec(memory_space=pltpu.VMEM),
            pl.BlockSpec(memory_space=pltpu.VMEM),
            pl.BlockSpec(memory_space=pltpu.MemorySpace.HBM),
            pl.BlockSpec(memory_space=pltpu.MemorySpace.HBM),
            pl.BlockSpec(memory_space=pltpu.VMEM),
        ],
        out_specs=pl.BlockSpec(memory_space=pltpu.VMEM),
        scratch_shapes=[
            pltpu.VMEM((N_DEV, HG, DH, DM), jnp.bfloat16),
            pltpu.VMEM((N_DEV, HG, DH, DM), jnp.bfloat16),
            pltpu.VMEM((2, HG, SKV, DH), jnp.float32),
            pltpu.VMEM((2, HG, SKV, DH), jnp.float32),
            pltpu.SemaphoreType.DMA((N_DEV - 1,)),
            pltpu.SemaphoreType.DMA((N_DEV - 1,)),
            pltpu.SemaphoreType.DMA((N_DEV - 1,)),
            pltpu.SemaphoreType.DMA((N_DEV - 1,)),
            pltpu.SemaphoreType.DMA((2,)),
            pltpu.SemaphoreType.DMA((2,)),
        ],
        compiler_params=pltpu.CompilerParams(collective_id=0),
    )(x, Wq, K_ext, V_ext, Wo)


# device time: 50197 ns/iter; 4.8752x vs baseline; 3.9437x over previous
import jax
import jax.numpy as jnp
from jax import lax
from jax.experimental import pallas as pl
from jax.experimental.pallas import tpu as pltpu

N_DEV = 4
B_LOC = 2
SQ = 512
SKV = 512
HG = 8
DH = 64
DM = 768
DG = HG * DH

NEG = -1e9


def kernel(x, Wq, K_ext, V_ext, Wo):
    my_i = lax.axis_index("i")

    x = x.astype(jnp.bfloat16)
    Wq = (Wq * 0.125).astype(jnp.bfloat16)
    Wo = Wo.astype(jnp.bfloat16)

    def prep(a):
        a = lax.dynamic_slice_in_dim(a, my_i * B_LOC, B_LOC, axis=0)
        return a.astype(jnp.bfloat16).transpose(0, 2, 3, 1)

    Kt = prep(K_ext)
    Vt = prep(V_ext)
    Vt = jnp.concatenate(
        [Vt,
         jnp.ones((B_LOC, N_DEV * HG, 1, SKV), jnp.bfloat16),
         jnp.zeros((B_LOC, N_DEV * HG, 7, SKV), jnp.bfloat16)], axis=2)

    def body(x_ref, wq_ref, kt_ref, vt_ref, wo_ref, out_ref,
             wq_s, wo_s, ctx_tmp, ctx_hold,
             wq_send, wq_recv, wo_send, wo_recv):
        me = lax.axis_index("i")
        left = (me + N_DEV - 1) % N_DEV
        right = (me + 1) % N_DEV

        barrier = pltpu.get_barrier_semaphore()
        for nbr in (left, right):
            pl.semaphore_signal(barrier, inc=1, device_id=(nbr,),
                                device_id_type=pl.DeviceIdType.MESH)
        pl.semaphore_wait(barrier, 2)

        wq_s[0] = wq_ref[...]
        wo_s[0] = wo_ref[...]

        HF = DG // 2

        def make_hop(h, half):
            c0, c1 = half * HF, (half + 1) * HF
            cp_q = pltpu.make_async_remote_copy(
                src_ref=wq_s.at[h - 1, :, pl.ds(c0, HF)],
                dst_ref=wq_s.at[h, :, pl.ds(c0, HF)],
                send_sem=wq_send.at[h - 1, half],
                recv_sem=wq_recv.at[h - 1, half],
                device_id=(right,), device_id_type=pl.DeviceIdType.MESH)
            cp_o = pltpu.make_async_remote_copy(
                src_ref=wo_s.at[h - 1, pl.ds(c0, HF), :],
                dst_ref=wo_s.at[h, pl.ds(c0, HF), :],
                send_sem=wo_send.at[h - 1, half],
                recv_sem=wo_recv.at[h - 1, half],
                device_id=(left,), device_id_type=pl.DeviceIdType.MESH)
            cp_q.start()
            cp_o.start()
            return (cp_q, cp_o)

        hops = {(1, 0): make_hop(1, 0), (1, 1): make_hop(1, 1)}

        def advance(h, half):
            cp_q, cp_o = hops[(h, half)]
            cp_q.wait()
            cp_o.wait()
            if h + 1 < N_DEV:
                hops[(h + 1, half)] = make_hop(h + 1, half)

        qi = lax.broadcasted_iota(jnp.int32, (SQ, SKV), 0)
        ki = lax.broadcasted_iota(jnp.int32, (SQ, SKV), 1)
        mask = (jnp.abs(qi - ki) <= 128) | (ki < 32) | (qi < 32)
        maskf = jnp.where(mask, 0.0, NEG).astype(jnp.float32)

        def attention(s, b, dst_ref):
            g = (me + N_DEV - s) % N_DEV
            q = lax.dot_general(
                x_ref[b], wq_s[s], (((1,), (0,)), ((), ())),
                preferred_element_type=jnp.float32)
            q = q.astype(jnp.bfloat16)
            for hh in range(HG):
                h_idx = g * HG + hh
                qh = q[:, hh * DH:(hh + 1) * DH]
                scores = lax.dot_general(
                    qh, kt_ref[b, h_idx], (((1,), (0,)), ((), ())),
                    preferred_element_type=jnp.float32)
                w = jnp.exp(scores + maskf)
                ctx = lax.dot_general(
                    w.astype(jnp.bfloat16), vt_ref[b, h_idx],
                    (((1,), (1,)), ((), ())),
                    preferred_element_type=jnp.float32)
                wsum = ctx[:, DH:DH + 1]
                dst_ref[b, :, hh * DH:(hh + 1) * DH] = (
                    ctx[:, :DH] / wsum).astype(jnp.bfloat16)

        def project(wo_slot, b, src_ref, first):
            part = lax.dot_general(
                src_ref[b], wo_s[wo_slot], (((1,), (0,)), ((), ())),
                preferred_element_type=jnp.float32)
            if first:
                out_ref[b] = part
            else:
                out_ref[b] = out_ref[b] + part

        attention(0, 0, ctx_tmp)
        project(0, 0, ctx_tmp, first=True)
        advance(1, 0)
        attention(0, 1, ctx_tmp)
        project(0, 1, ctx_tmp, first=True)
        advance(1, 1)

        attention(1, 0, ctx_hold)
        advance(2, 0)
        attention(1, 1, ctx_hold)
        advance(2, 1)

        attention(2, 0, ctx_tmp)
        project(2, 0, ctx_tmp, first=False)
        advance(3, 0)
        attention(2, 1, ctx_tmp)
        project(2, 1, ctx_tmp, first=False)
        advance(3, 1)

        for b in range(B_LOC):
            attention(3, b, ctx_tmp)
            project(1, b, ctx_tmp, first=False)
        for b in range(B_LOC):
            project(3, b, ctx_hold, first=False)

    return pl.pallas_call(
        body,
        out_shape=jax.ShapeDtypeStruct((B_LOC, SQ, DM), jnp.float32),
        in_specs=[
            pl.BlockSpec(memory_space=pltpu.VMEM),
            pl.BlockSpec(memory_space=pltpu.VMEM),
            pl.BlockSpec(memory_space=pltpu.VMEM),
            pl.BlockSpec(memory_space=pltpu.VMEM),
            pl.BlockSpec(memory_space=pltpu.VMEM),
        ],
        out_specs=pl.BlockSpec(memory_space=pltpu.VMEM),
        scratch_shapes=[
            pltpu.VMEM((N_DEV, DM, DG), jnp.bfloat16),
            pltpu.VMEM((N_DEV, DG, DM), jnp.bfloat16),
            pltpu.VMEM((B_LOC, SQ, DG), jnp.bfloat16),
            pltpu.VMEM((B_LOC, SQ, DG), jnp.bfloat16),
            pltpu.SemaphoreType.DMA((N_DEV - 1, 2)),
            pltpu.SemaphoreType.DMA((N_DEV - 1, 2)),
            pltpu.SemaphoreType.DMA((N_DEV - 1, 2)),
            pltpu.SemaphoreType.DMA((N_DEV - 1, 2)),
        ],
        compiler_params=pltpu.CompilerParams(collective_id=0),
    )(x, Wq, Kt, Vt, Wo)
